# baseline (device time: 88760 ns/iter reference)
import os

import jax
import jax.numpy as jnp
from jax import lax
from jax.experimental import pallas as pl
from jax.experimental.pallas import tpu as pltpu

_KMODE = os.environ.get("KMODE", "full")

H = 16
S_PER = 1024
D = 128
SCALE = D ** -0.5
H_MINE = H // 2
GROUP = 2
N_FLOWS = H_MINE // GROUP


def _compute(q, kv, kvo):
    qb = (q * SCALE).astype(jnp.bfloat16)
    s1 = lax.dot_general(
        qb, kv[:, :D], (((1,), (1,)), ((), ())), preferred_element_type=jnp.float32
    )
    s2 = lax.dot_general(
        qb, kvo[:, :D], (((1,), (1,)), ((), ())), preferred_element_type=jnp.float32
    )
    p1 = jnp.exp(s1).astype(jnp.bfloat16)
    p2 = jnp.exp(s2).astype(jnp.bfloat16)
    ones = jnp.ones((S_PER, 1), jnp.bfloat16)
    l = lax.dot_general(
        p1, ones, (((1,), (0,)), ((), ())), preferred_element_type=jnp.float32
    ) + lax.dot_general(
        p2, ones, (((1,), (0,)), ((), ())), preferred_element_type=jnp.float32
    )
    o = lax.dot_general(
        p1, kv[:, D:], (((1,), (0,)), ((), ())), preferred_element_type=jnp.float32
    ) + lax.dot_general(
        p2, kvo[:, D:], (((1,), (0,)), ((), ())), preferred_element_type=jnp.float32
    )
    return (o / l).astype(jnp.bfloat16)


def _body(q_ref, kvsm_ref, out_ref, kv_ref, kvo_ref, dsend, drecv, osend, orecv):
    s = pl.program_id(0)
    my_x = lax.axis_index("x")
    my_y = lax.axis_index("y")
    ynbr = (my_x, 1 - my_y)
    xnbr = (1 - my_x, my_y)
    base = my_x * H_MINE

    if _KMODE == "compute":
        @pl.when(s == 0)
        def _relayout():
            for h in range(H_MINE):
                kv_ref[h] = kvsm_ref[:, h, :]

        out_ref[base + s] = _compute(q_ref[0], kv_ref[s], kv_ref[s])
        return

    def kv_flow(g):
        return pltpu.make_async_remote_copy(
            src_ref=kv_ref.at[pl.ds(GROUP * g, GROUP)],
            dst_ref=kvo_ref.at[pl.ds(GROUP * g, GROUP)],
            send_sem=dsend.at[g],
            recv_sem=drecv.at[g],
            device_id=ynbr,
            device_id_type=pl.DeviceIdType.MESH,
        )

    def out_flow(g):
        slc = out_ref.at[pl.ds(base + GROUP * g, GROUP)]
        return pltpu.make_async_remote_copy(
            src_ref=slc,
            dst_ref=slc,
            send_sem=osend.at[g],
            recv_sem=orecv.at[g],
            device_id=xnbr,
            device_id_type=pl.DeviceIdType.MESH,
        )

    @pl.when(s == 0)
    def _start():
        barrier_sem = pltpu.get_barrier_semaphore()
        for nbr in (ynbr, xnbr):
            pl.semaphore_signal(
                barrier_sem, inc=1, device_id=nbr,
                device_id_type=pl.DeviceIdType.MESH,
            )
        pl.semaphore_wait(barrier_sem, 2)
        for g in range(N_FLOWS):
            for h in range(GROUP * g, GROUP * (g + 1)):
                kv_ref[h] = kvsm_ref[:, h, :]
            kv_flow(g).start()

    @pl.when(s % GROUP == 0)
    def _wait_group():
        g = s // GROUP
        kv_flow(g).wait_recv()
        kv_flow(g).wait_send()

    if _KMODE == "comm":
        out_ref[base + s] = kvo_ref[s][:, :D]
    else:
        out_ref[base + s] = _compute(q_ref[0], kv_ref[s], kvo_ref[s])

    @pl.when(s % GROUP == GROUP - 1)
    def _ship_group():
        out_flow(s // GROUP).start()

    @pl.when(s == H_MINE - 1)
    def _finish():
        for g in range(N_FLOWS):
            out_flow(g).wait_send()
            out_flow(g).wait_recv()


def kernel(Q, K, V):
    my_x = lax.axis_index("x")
    base_h = my_x * H_MINE
    qm = jnp.transpose(
        lax.dynamic_slice(Q[0], (0, base_h, 0), (S_PER, H_MINE, D)), (1, 0, 2)
    )
    kv = jnp.concatenate(
        [
            lax.dynamic_slice(K[0], (0, base_h, 0), (S_PER, H_MINE, D)),
            lax.dynamic_slice(V[0], (0, base_h, 0), (S_PER, H_MINE, D)),
        ],
        axis=-1,
    ).astype(jnp.bfloat16)

    out = pl.pallas_call(
        _body,
        grid=(H_MINE,),
        out_shape=jax.ShapeDtypeStruct((H, S_PER, D), jnp.bfloat16),
        in_specs=[
            pl.BlockSpec((1, S_PER, D), lambda s: (s, 0, 0)),
            pl.BlockSpec(memory_space=pltpu.VMEM),
        ],
        out_specs=pl.BlockSpec(memory_space=pltpu.VMEM),
        scratch_shapes=[
            pltpu.VMEM((H_MINE, S_PER, 2 * D), jnp.bfloat16),
            pltpu.VMEM((H_MINE, S_PER, 2 * D), jnp.bfloat16),
            pltpu.SemaphoreType.DMA((N_FLOWS,)),
            pltpu.SemaphoreType.DMA((N_FLOWS,)),
            pltpu.SemaphoreType.DMA((N_FLOWS,)),
            pltpu.SemaphoreType.DMA((N_FLOWS,)),
        ],
        compiler_params=pltpu.CompilerParams(
            collective_id=None if _KMODE == "compute" else 0,
            vmem_limit_bytes=50 * 1024 * 1024,
        ),
    )(qm, kv)

    return jnp.transpose(out, (1, 0, 2))[None]


# device time: 87693 ns/iter; 1.0122x vs baseline; 1.0122x over previous
import os

import jax
import jax.numpy as jnp
from jax import lax
from jax.experimental import pallas as pl
from jax.experimental.pallas import tpu as pltpu

_KMODE = os.environ.get("KMODE", "full")

H = 16
S_PER = 1024
D = 128
SCALE = D ** -0.5
H_MINE = H // 2
GROUP = 2
N_FLOWS = H_MINE // GROUP


def _compute(q, kv, kvo):
    qb = (q * SCALE).astype(jnp.bfloat16)
    s1 = lax.dot_general(
        qb, kv[:, :D], (((1,), (1,)), ((), ())), preferred_element_type=jnp.float32
    )
    s2 = lax.dot_general(
        qb, kvo[:, :D], (((1,), (1,)), ((), ())), preferred_element_type=jnp.float32
    )
    p1 = jnp.exp(s1).astype(jnp.bfloat16)
    p2 = jnp.exp(s2).astype(jnp.bfloat16)
    ones = jnp.ones((S_PER, 1), jnp.bfloat16)
    l = lax.dot_general(
        p1, ones, (((1,), (0,)), ((), ())), preferred_element_type=jnp.float32
    ) + lax.dot_general(
        p2, ones, (((1,), (0,)), ((), ())), preferred_element_type=jnp.float32
    )
    o = lax.dot_general(
        p1, kv[:, D:], (((1,), (0,)), ((), ())), preferred_element_type=jnp.float32
    ) + lax.dot_general(
        p2, kvo[:, D:], (((1,), (0,)), ((), ())), preferred_element_type=jnp.float32
    )
    return (o / l).astype(jnp.bfloat16)


def _body(q_ref, kv_ref, out_ref, kvo_ref, dsend, drecv, osend, orecv):
    s = pl.program_id(0)
    my_x = lax.axis_index("x")
    my_y = lax.axis_index("y")
    ynbr = (my_x, 1 - my_y)
    xnbr = (1 - my_x, my_y)
    base = my_x * H_MINE

    if _KMODE == "compute":
        out_ref[base + s] = _compute(q_ref[0], kv_ref[s], kv_ref[s])
        return

    def kv_flow(g):
        return pltpu.make_async_remote_copy(
            src_ref=kv_ref.at[pl.ds(GROUP * g, GROUP)],
            dst_ref=kvo_ref.at[pl.ds(GROUP * g, GROUP)],
            send_sem=dsend.at[g],
            recv_sem=drecv.at[g],
            device_id=ynbr,
            device_id_type=pl.DeviceIdType.MESH,
        )

    def out_flow(g):
        slc = out_ref.at[pl.ds(base + GROUP * g, GROUP)]
        return pltpu.make_async_remote_copy(
            src_ref=slc,
            dst_ref=slc,
            send_sem=osend.at[g],
            recv_sem=orecv.at[g],
            device_id=xnbr,
            device_id_type=pl.DeviceIdType.MESH,
        )

    @pl.when(s == 0)
    def _start():
        barrier_sem = pltpu.get_barrier_semaphore()
        for nbr in (ynbr, xnbr):
            pl.semaphore_signal(
                barrier_sem, inc=1, device_id=nbr,
                device_id_type=pl.DeviceIdType.MESH,
            )
        pl.semaphore_wait(barrier_sem, 2)
        for g in range(N_FLOWS):
            kv_flow(g).start()

    @pl.when(s % GROUP == 0)
    def _wait_group():
        g = s // GROUP
        kv_flow(g).wait_recv()
        kv_flow(g).wait_send()

    if _KMODE == "comm":
        out_ref[base + s] = kvo_ref[s][:, :D]
    else:
        out_ref[base + s] = _compute(q_ref[0], kv_ref[s], kvo_ref[s])

    @pl.when(s % GROUP == GROUP - 1)
    def _ship_group():
        out_flow(s // GROUP).start()

    @pl.when(s == H_MINE - 1)
    def _finish():
        for g in range(N_FLOWS):
            out_flow(g).wait_send()
            out_flow(g).wait_recv()


def kernel(Q, K, V):
    my_x = lax.axis_index("x")
    base_h = my_x * H_MINE
    qm = jnp.transpose(
        lax.dynamic_slice(Q[0], (0, base_h, 0), (S_PER, H_MINE, D)), (1, 0, 2)
    )
    kv = jnp.transpose(
        jnp.concatenate(
            [
                lax.dynamic_slice(K[0], (0, base_h, 0), (S_PER, H_MINE, D)),
                lax.dynamic_slice(V[0], (0, base_h, 0), (S_PER, H_MINE, D)),
            ],
            axis=-1,
        ).astype(jnp.bfloat16),
        (1, 0, 2),
    )

    out = pl.pallas_call(
        _body,
        grid=(H_MINE,),
        out_shape=jax.ShapeDtypeStruct((H, S_PER, D), jnp.bfloat16),
        in_specs=[
            pl.BlockSpec((1, S_PER, D), lambda s: (s, 0, 0)),
            pl.BlockSpec(memory_space=pltpu.VMEM),
        ],
        out_specs=pl.BlockSpec(memory_space=pltpu.VMEM),
        scratch_shapes=[
            pltpu.VMEM((H_MINE, S_PER, 2 * D), jnp.bfloat16),
            pltpu.SemaphoreType.DMA((N_FLOWS,)),
            pltpu.SemaphoreType.DMA((N_FLOWS,)),
            pltpu.SemaphoreType.DMA((N_FLOWS,)),
            pltpu.SemaphoreType.DMA((N_FLOWS,)),
        ],
        compiler_params=pltpu.CompilerParams(
            collective_id=None if _KMODE == "compute" else 0,
            vmem_limit_bytes=50 * 1024 * 1024,
        ),
    )(qm, kv)

    return jnp.transpose(out, (1, 0, 2))[None]
